# baseline (speedup 1.0000x reference)
import functools

import jax
import jax.numpy as jnp
from jax import lax
from jax.experimental import pallas as pl
from jax.experimental.pallas import tpu as pltpu
from jax.experimental.pallas import tpu_sc as plsc

VOCAB_SIZE = 1000000
EMBED_DIM = 64
BATCH = 4096
LENGTH = 50

_INFO = plsc.get_sparse_core_info()
NC = _INFO.num_cores
NS = _INFO.num_subcores
NW = NC * NS
B_TOTAL = BATCH * LENGTH
CHUNK = 128
CHUNKS_TOTAL = B_TOTAL // CHUNK
CPW = CHUNKS_TOTAL // NW
BPW = CPW * CHUNK
GROUPS = CHUNK // 16

BLK = 128
FULL_BLOCKS = VOCAB_SIZE // BLK
BLOCKS_PER_W = -(-FULL_BLOCKS // NW)
PACKED_ROWS = (FULL_BLOCKS + 1) * (BLK // 2)

_PARAMS = pltpu.CompilerParams(use_tc_tiling_on_sc=True,
                               needs_layout_passes=False)
_MESH = plsc.VectorSubcoreMesh(core_axis_name="c", subcore_axis_name="s")


def _sc_gather(idx_hbm, packed_hbm):
    @functools.partial(
        pl.kernel,
        out_type=jax.ShapeDtypeStruct((LENGTH, EMBED_DIM, BATCH), jnp.float32),
        mesh=_MESH,
        scratch_types=[
            pltpu.VMEM((BPW,), jnp.int32),
            pltpu.VMEM((BPW,), jnp.int32),
            pltpu.VMEM((BPW,), jnp.int32),
            pltpu.VMEM((2, CHUNK, 2 * EMBED_DIM), jnp.float32),
            pltpu.VMEM((2, EMBED_DIM, CHUNK), jnp.float32),
            [pltpu.SemaphoreType.DMA] * 2,
            [pltpu.SemaphoreType.DMA] * 2,
        ],
        compiler_params=_PARAMS,
    )
    def k(idx_ref, packed_ref, out_ref, idx_v, rid_v, par_v, rows, trows,
          gsems, ssems):
        wid = lax.axis_index("s") * NC + lax.axis_index("c")
        base = wid * BPW
        pltpu.sync_copy(idx_ref.at[pl.ds(base, BPW)], idx_v)

        @pl.loop(0, BPW // 16)
        def split(g):
            v = idx_v[pl.ds(g * 16, 16)]
            par = jnp.where(v >= (VOCAB_SIZE // 2), 1, 0).astype(jnp.int32)
            rid_v[pl.ds(g * 16, 16)] = v - par * (VOCAB_SIZE // 2)
            par_v[pl.ds(g * 16, 16)] = par

        def gather(c, b):
            rid = rid_v.at[pl.ds(c * CHUNK, CHUNK)]
            pltpu.async_copy(packed_ref.at[rid], rows.at[b], gsems[b])

        def wait_gather(b):
            pltpu.make_async_copy(
                packed_ref.at[pl.ds(0, CHUNK)], rows.at[b], gsems[b]
            ).wait()

        def out_slice(c):
            k0 = base + c * CHUNK
            return out_ref.at[k0 // BATCH, :, pl.ds(k0 % BATCH, CHUNK)]

        def wait_scatter(b):
            pltpu.make_async_copy(trows.at[b], out_slice(0), ssems[b]).wait()

        iota = lax.iota(jnp.int32, 16)

        def transpose_chunk(c, b):
            for g in range(GROUPS):
                pvec = par_v[pl.ds(c * CHUNK + g * 16, 16)]
                jrow = iota + (g * 16)
                pbase = pvec * EMBED_DIM
                for d in range(EMBED_DIM):
                    vals = plsc.load_gather(rows.at[b], [jrow, pbase + d])
                    trows[b, d, pl.ds(g * 16, 16)] = vals

        gather(0, 0)
        gather(1, 1)

        @pl.loop(0, CPW, step=2)
        def pipelined(j):
            for b in range(2):
                c = j + b
                wait_gather(b)
                transpose_chunk(c, b)

                @pl.when(c + 2 < CPW)
                def _():
                    gather(c + 2, b)

                @pl.when(j > 0)
                def _():
                    wait_scatter(b)

                pltpu.async_copy(trows.at[b], out_slice(c), ssems[b])

        wait_scatter(0)
        wait_scatter(1)

    return k(idx_hbm, packed_hbm)


def kernel(src_sents, weight):
    idx = src_sents.astype(jnp.int32).T.reshape(B_TOTAL)
    packed = jnp.concatenate(
        [weight[: VOCAB_SIZE // 2], weight[VOCAB_SIZE // 2:]], axis=1)
    out = _sc_gather(idx, packed)
    return out.transpose(2, 0, 1)


# speedup vs baseline: 1.4127x; 1.4127x over previous
import functools

import jax
import jax.numpy as jnp
from jax import lax
from jax.experimental import pallas as pl
from jax.experimental.pallas import tpu as pltpu
from jax.experimental.pallas import tpu_sc as plsc

VOCAB_SIZE = 1000000
EMBED_DIM = 64
BATCH = 4096
LENGTH = 50

_INFO = plsc.get_sparse_core_info()
NC = _INFO.num_cores
NS = _INFO.num_subcores
NW = NC * NS
B_TOTAL = BATCH * LENGTH
CHUNK = 128
CHUNKS_TOTAL = B_TOTAL // CHUNK
CPW = CHUNKS_TOTAL // NW
BPW = CPW * CHUNK


def _sc_gather(idx_hbm, table_hbm):
    mesh = plsc.VectorSubcoreMesh(core_axis_name="c", subcore_axis_name="s")

    nbuf = 5

    @functools.partial(
        pl.kernel,
        out_type=jax.ShapeDtypeStruct((B_TOTAL, EMBED_DIM), jnp.float32),
        mesh=mesh,
        scratch_types=[
            pltpu.VMEM((BPW,), jnp.int32),
            pltpu.VMEM((nbuf, CHUNK, EMBED_DIM), jnp.float32),
            [pltpu.SemaphoreType.DMA] * nbuf,
            [pltpu.SemaphoreType.DMA] * nbuf,
        ],
        compiler_params=pltpu.CompilerParams(use_tc_tiling_on_sc=False),
    )
    def k(idx_ref, table_ref, out_ref, idx_v, rows, gsems, ssems):
        wid = lax.axis_index("s") * NC + lax.axis_index("c")
        base = wid * BPW
        pltpu.sync_copy(idx_ref.at[pl.ds(base, BPW)], idx_v)

        def gather(c, b):
            chunk_idx = idx_v.at[pl.ds(c * CHUNK, CHUNK)]
            pltpu.async_copy(table_ref.at[chunk_idx], rows.at[b], gsems[b])

        def out_slice(c):
            return out_ref.at[pl.ds(base + c * CHUNK, CHUNK)]

        def wait_gather(b):
            pltpu.make_async_copy(
                table_ref.at[pl.ds(0, CHUNK)], rows.at[b], gsems[b]
            ).wait()

        def wait_scatter(b):
            pltpu.make_async_copy(rows.at[b], out_slice(0), ssems[b]).wait()

        for b in range(nbuf):
            gather(b, b)

        @pl.loop(0, CPW - nbuf, step=nbuf)
        def pipelined(j):
            for b in range(nbuf):
                wait_gather(b)
                pltpu.async_copy(rows.at[b], out_slice(j + b), ssems[b])
            for b in range(nbuf):
                wait_scatter(b)
                gather(j + nbuf + b, b)

        for b in range(nbuf):
            wait_gather(b)
            pltpu.async_copy(rows.at[b], out_slice(CPW - nbuf + b), ssems[b])
        for b in range(nbuf):
            wait_scatter(b)

    return k(idx_hbm, table_hbm)


def kernel(src_sents, weight):
    idx = src_sents.astype(jnp.int32).T.reshape(B_TOTAL)
    out = _sc_gather(idx, weight)
    return out.reshape(LENGTH, BATCH, EMBED_DIM).transpose(1, 0, 2)
